# baseline (speedup 1.0000x reference)
import functools

import jax
import jax.numpy as jnp
from jax import lax
from jax.experimental import pallas as pl
from jax.experimental.pallas import tpu as pltpu
from jax.experimental.pallas import tpu_sc as plsc

NC = 16384
EC = 4096
ROWS_N = 2048
ROWS_E = 1024
NW = 32
F = 128

_MESH = dict(core_axis_name="c", subcore_axis_name="s")


def _wid():
    return lax.axis_index("s") * 2 + lax.axis_index("c")


def _mk_gather1(M, T):
    rw = M // NW
    ch = min(rw, 512)

    @functools.partial(
        pl.kernel,
        out_type=jax.ShapeDtypeStruct((M, F), jnp.float32),
        mesh=plsc.VectorSubcoreMesh(**_MESH),
        scratch_types=[
            pltpu.VMEM((ch,), jnp.int32),
            pltpu.VMEM((ch, F), jnp.float32),
            pltpu.SemaphoreType.DMA,
        ],
    )
    def k(tab_ref, idx_ref, out_ref, idxv, rowsv, sem):
        base = _wid() * rw
        for j in range(rw // ch):
            o = base + j * ch
            pltpu.sync_copy(idx_ref.at[pl.ds(o, ch)], idxv)
            pltpu.async_copy(tab_ref.at[idxv], rowsv, sem).wait()
            pltpu.sync_copy(rowsv, out_ref.at[pl.ds(o, ch)])

    return k


def _mk_gather2(M):
    rw = M // NW
    ch = min(rw, 512)

    @functools.partial(
        pl.kernel,
        out_type=[jax.ShapeDtypeStruct((M, F), jnp.float32),
                  jax.ShapeDtypeStruct((M, F), jnp.float32)],
        mesh=plsc.VectorSubcoreMesh(**_MESH),
        scratch_types=[
            pltpu.VMEM((ch,), jnp.int32),
            pltpu.VMEM((ch, F), jnp.float32),
            pltpu.VMEM((ch, F), jnp.float32),
            pltpu.SemaphoreType.DMA,
        ],
    )
    def k(ta_ref, tb_ref, idx_ref, oa_ref, ob_ref, idxv, ra, rb, sem):
        base = _wid() * rw
        for j in range(rw // ch):
            o = base + j * ch
            pltpu.sync_copy(idx_ref.at[pl.ds(o, ch)], idxv)
            pltpu.async_copy(ta_ref.at[idxv], ra, sem).wait()
            pltpu.async_copy(tb_ref.at[idxv], rb, sem).wait()
            pltpu.sync_copy(ra, oa_ref.at[pl.ds(o, ch)])
            pltpu.sync_copy(rb, ob_ref.at[pl.ds(o, ch)])

    return k


def _mk_update_hc(M):
    rw = M // NW
    ch = min(rw, 512)

    @functools.partial(
        pl.kernel,
        out_type=[],
        mesh=plsc.VectorSubcoreMesh(**_MESH),
        scratch_types=[
            pltpu.VMEM((ch,), jnp.int32),
            pltpu.VMEM((ch, F), jnp.float32),
            pltpu.SemaphoreType.DMA,
        ],
    )
    def k(nh_ref, ncl_ref, dst_ref, h_ref, c_ref, idxv, rowsv, sem):
        base = _wid() * rw
        for j in range(rw // ch):
            o = base + j * ch
            pltpu.sync_copy(dst_ref.at[pl.ds(o, ch)], idxv)
            pltpu.sync_copy(nh_ref.at[pl.ds(o, ch)], rowsv)
            pltpu.async_copy(rowsv, h_ref.at[idxv], sem).wait()
            pltpu.sync_copy(ncl_ref.at[pl.ds(o, ch)], rowsv)
            pltpu.async_copy(rowsv, c_ref.at[idxv], sem).wait()

    return k


def _stage1n_body(x_ref, flat_ref, wiou_ref, uiou_ref, biou_ref, iu_ref, o_ref):
    out_f = iu_ref.shape[-1]
    iou = (jnp.dot(x_ref[...], wiou_ref[...], preferred_element_type=jnp.float32)
           + jnp.dot(flat_ref[...], uiou_ref[...], preferred_element_type=jnp.float32)
           + biou_ref[...])
    i = jax.nn.sigmoid(iou[:, :out_f])
    o = jax.nn.sigmoid(iou[:, out_f:2 * out_f])
    u = jnp.tanh(iou[:, 2 * out_f:])
    iu_ref[...] = i * u
    o_ref[...] = o


def _stage1e_body(xe_ref, ch_ref, cc_ref, wf_ref, uf_ref, bf_ref, fc_ref):
    f = jax.nn.sigmoid(
        jnp.dot(xe_ref[...], wf_ref[...], preferred_element_type=jnp.float32)
        + jnp.dot(ch_ref[...], uf_ref[...], preferred_element_type=jnp.float32)
        + bf_ref[...])
    fc_ref[...] = f * cc_ref[...]


def _stage2_body(flat_ref, wc_ref, bc_ref, iu_ref, o_ref, h_ref, c_ref):
    cr = jnp.dot(flat_ref[...], wc_ref[...],
                 preferred_element_type=jnp.float32) + bc_ref[...]
    nc = iu_ref[...] + cr
    c_ref[...] = nc
    h_ref[...] = o_ref[...] * jnp.tanh(nc)


def kernel(forest, adjacency, node_order, edge_order, W_iou_w, W_iou_b,
           U_iou_w, W_c_w, W_c_b, W_f_w, W_f_b, U_f_w):
    N, in_f = forest.shape
    out_f = W_f_w.shape[0]
    trip = 3 * out_f
    E = adjacency.shape[0]
    max_it = 4
    bf3 = 3

    parent = adjacency[:, 0]
    child = adjacency[:, 1]
    slot = jnp.clip(adjacency[:, 2] + 1, 0, bf3 - 1)
    valid = (parent >= 0) & (parent < N) & (child >= 0) & (child < N)
    sp = jnp.clip(parent, 0, N - 1)
    sc_ = jnp.clip(child, 0, N - 1)

    wiou_t = W_iou_w.T
    uiou_t = U_iou_w.T
    wc_t = W_c_w.T
    wf_t = W_f_w.T
    uf_t = U_f_w.T
    b_iou = W_iou_b.reshape(1, trip)
    b_c = W_c_b.reshape(1, out_f)
    b_f = W_f_b.reshape(1, out_f)

    node_iter_of_parent = node_order[sp]
    arangeN = jnp.arange(N, dtype=jnp.int32)
    arangeE = jnp.arange(E, dtype=jnp.int32)
    inv_pos = jnp.zeros((N,), jnp.int32)
    active, counts, elists, ecounts = [], [], [], []
    for t in range(max_it):
        mask = node_order == t
        r = jnp.cumsum(mask.astype(jnp.int32)) - 1
        inv_pos = inv_pos + jnp.where(mask, r, 0)
        dest = jnp.where(mask, r, NC)
        active.append(jnp.zeros((NC,), jnp.int32).at[dest].set(
            arangeN, mode="drop"))
        counts.append(jnp.sum(mask.astype(jnp.int32)))
        emask = valid & (edge_order == t) & (node_iter_of_parent == t)
        er = jnp.cumsum(emask.astype(jnp.int32)) - 1
        edest = jnp.where(emask, er, EC)
        elists.append(jnp.zeros((EC,), jnp.int32).at[edest].set(
            arangeE, mode="drop"))
        ecounts.append(jnp.sum(emask.astype(jnp.int32)))

    eids_all = jnp.concatenate(elists)
    cidx_all = sc_[eids_all].reshape(max_it, EC)
    pidx_all = sp[eids_all]
    dest3_all = inv_pos[pidx_all] * bf3 + slot[eids_all]
    ev_all = (jnp.tile(arangeE[:EC], max_it)
              < jnp.repeat(jnp.stack(ecounts), EC))
    dest3_all = jnp.where(ev_all, dest3_all, NC * bf3).reshape(max_it, EC)
    aid_all = jnp.concatenate(active)
    av_all = (jnp.tile(arangeN[:NC], max_it)
              < jnp.repeat(jnp.stack(counts), NC))
    hdest_all = jnp.where(av_all, aid_all, N).reshape(max_it, NC)

    gather_x = _mk_gather1(max_it * NC, N)
    gather_xe = _mk_gather1(max_it * EC, N)
    gather_hc = _mk_gather2(EC)
    update_hc = _mk_update_hc(NC)

    x_all = gather_x(forest, aid_all)
    xe_all = gather_xe(forest, pidx_all)

    g_n = NC // ROWS_N
    g_e = EC // ROWS_E
    full = lambda i: (0, 0)
    blk = lambda i: (i, 0)

    def mk_stage1n(t):
        xoff = t * g_n
        return pl.pallas_call(
            _stage1n_body,
            grid=(g_n,),
            in_specs=[
                pl.BlockSpec((ROWS_N, in_f), lambda i: (xoff + i, 0)),
                pl.BlockSpec((ROWS_N, trip), blk),
                pl.BlockSpec((in_f, trip), full),
                pl.BlockSpec((trip, trip), full),
                pl.BlockSpec((1, trip), full),
            ],
            out_specs=[
                pl.BlockSpec((ROWS_N, out_f), blk),
                pl.BlockSpec((ROWS_N, out_f), blk),
            ],
            out_shape=[
                jax.ShapeDtypeStruct((NC, out_f), jnp.float32),
                jax.ShapeDtypeStruct((NC, out_f), jnp.float32),
            ],
        )

    def mk_stage1e(t):
        xoff = t * g_e
        return pl.pallas_call(
            _stage1e_body,
            grid=(g_e,),
            in_specs=[
                pl.BlockSpec((ROWS_E, in_f), lambda i: (xoff + i, 0)),
                pl.BlockSpec((ROWS_E, out_f), blk),
                pl.BlockSpec((ROWS_E, out_f), blk),
                pl.BlockSpec((in_f, out_f), full),
                pl.BlockSpec((out_f, out_f), full),
                pl.BlockSpec((1, out_f), full),
            ],
            out_specs=pl.BlockSpec((ROWS_E, out_f), blk),
            out_shape=jax.ShapeDtypeStruct((EC, out_f), jnp.float32),
        )

    stage2 = pl.pallas_call(
        _stage2_body,
        grid=(g_n,),
        in_specs=[
            pl.BlockSpec((ROWS_N, trip), blk),
            pl.BlockSpec((trip, out_f), full),
            pl.BlockSpec((1, out_f), full),
            pl.BlockSpec((ROWS_N, out_f), blk),
            pl.BlockSpec((ROWS_N, out_f), blk),
        ],
        out_specs=[
            pl.BlockSpec((ROWS_N, out_f), blk),
            pl.BlockSpec((ROWS_N, out_f), blk),
        ],
        out_shape=[
            jax.ShapeDtypeStruct((NC, out_f), jnp.float32),
            jax.ShapeDtypeStruct((NC, out_f), jnp.float32),
        ],
    )

    h_ref = jax.new_ref(jnp.zeros((N + 16, out_f), jnp.float32))
    c_ref = jax.new_ref(jnp.zeros((N + 16, out_f), jnp.float32))

    for t in range(max_it):
        ch, cc = gather_hc(h_ref, c_ref, cidx_all[t])
        dest3 = dest3_all[t]
        flat_h = (jnp.zeros((NC * bf3, out_f), jnp.float32)
                  .at[dest3].add(ch, mode="drop").reshape(NC, trip))
        iu, o = mk_stage1n(t)(x_all, flat_h, wiou_t, uiou_t, b_iou)
        fc = mk_stage1e(t)(xe_all, ch, cc, wf_t, uf_t, b_f)
        flat_fc = (jnp.zeros((NC * bf3, out_f), jnp.float32)
                   .at[dest3].add(fc, mode="drop").reshape(NC, trip))
        nh, ncell = stage2(flat_fc, wc_t, b_c, iu, o)
        update_hc(nh, ncell, hdest_all[t], h_ref, c_ref)

    return h_ref[...][:N]


# speedup vs baseline: 1.0004x; 1.0004x over previous
import functools

import jax
import jax.numpy as jnp
from jax import lax
from jax.experimental import pallas as pl
from jax.experimental.pallas import tpu as pltpu
from jax.experimental.pallas import tpu_sc as plsc

NC = 16384
EC = 4096
ROWS_N = 2048
ROWS_E = 1024
NW = 32
F = 128

_MESH = dict(core_axis_name="c", subcore_axis_name="s")


def _wid():
    return lax.axis_index("s") * 2 + lax.axis_index("c")


def _mk_gather1(M, T, nbuf=4, ch=128):
    rw = M // NW
    ch = min(rw, ch)
    nch = rw // ch

    @functools.partial(
        pl.kernel,
        out_type=jax.ShapeDtypeStruct((M, F), jnp.float32),
        mesh=plsc.VectorSubcoreMesh(**_MESH),
        scratch_types=(
            [pltpu.VMEM((rw,), jnp.int32)]
            + [pltpu.VMEM((ch, F), jnp.float32) for _ in range(nbuf)]
            + [pltpu.SemaphoreType.DMA for _ in range(nbuf)]
            + [pltpu.SemaphoreType.DMA for _ in range(nbuf)]
        ),
    )
    def k(tab_ref, idx_ref, out_ref, *scr):
        idxv = scr[0]
        bufs = scr[1:1 + nbuf]
        gsems = scr[1 + nbuf:1 + 2 * nbuf]
        wsems = scr[1 + 2 * nbuf:1 + 3 * nbuf]
        base = _wid() * rw
        pltpu.sync_copy(idx_ref.at[pl.ds(base, rw)], idxv)
        gat = [None] * nbuf
        wb = [None] * nbuf

        def issue_wb(jj):
            bb = jj % nbuf
            gat[bb].wait()
            wb[bb] = pltpu.make_async_copy(
                bufs[bb], out_ref.at[pl.ds(base + jj * ch, ch)], wsems[bb])
            wb[bb].start()

        for j in range(nch):
            b = j % nbuf
            if wb[b] is not None:
                wb[b].wait()
            gat[b] = pltpu.make_async_copy(
                tab_ref.at[idxv.at[pl.ds(j * ch, ch)]], bufs[b], gsems[b])
            gat[b].start()
            if j >= nbuf - 1:
                issue_wb(j - (nbuf - 1))
        for jj in range(max(nch - (nbuf - 1), 0), nch):
            issue_wb(jj)
        for b in range(min(nbuf, nch)):
            wb[(nch - 1 - b) % nbuf].wait()

    return k


def _mk_gather2(M):
    rw = M // NW
    ch = min(rw, 512)

    @functools.partial(
        pl.kernel,
        out_type=[jax.ShapeDtypeStruct((M, F), jnp.float32),
                  jax.ShapeDtypeStruct((M, F), jnp.float32)],
        mesh=plsc.VectorSubcoreMesh(**_MESH),
        scratch_types=[
            pltpu.VMEM((ch,), jnp.int32),
            pltpu.VMEM((ch, F), jnp.float32),
            pltpu.VMEM((ch, F), jnp.float32),
            pltpu.SemaphoreType.DMA,
        ],
    )
    def k(ta_ref, tb_ref, idx_ref, oa_ref, ob_ref, idxv, ra, rb, sem):
        base = _wid() * rw
        for j in range(rw // ch):
            o = base + j * ch
            pltpu.sync_copy(idx_ref.at[pl.ds(o, ch)], idxv)
            pltpu.async_copy(ta_ref.at[idxv], ra, sem).wait()
            pltpu.async_copy(tb_ref.at[idxv], rb, sem).wait()
            pltpu.sync_copy(ra, oa_ref.at[pl.ds(o, ch)])
            pltpu.sync_copy(rb, ob_ref.at[pl.ds(o, ch)])

    return k


def _mk_update_hc(M):
    rw = M // NW
    ch = min(rw, 512)

    @functools.partial(
        pl.kernel,
        out_type=[],
        mesh=plsc.VectorSubcoreMesh(**_MESH),
        scratch_types=[
            pltpu.VMEM((ch,), jnp.int32),
            pltpu.VMEM((ch, F), jnp.float32),
            pltpu.SemaphoreType.DMA,
        ],
    )
    def k(nh_ref, ncl_ref, dst_ref, h_ref, c_ref, idxv, rowsv, sem):
        base = _wid() * rw
        for j in range(rw // ch):
            o = base + j * ch
            pltpu.sync_copy(dst_ref.at[pl.ds(o, ch)], idxv)
            pltpu.sync_copy(nh_ref.at[pl.ds(o, ch)], rowsv)
            pltpu.async_copy(rowsv, h_ref.at[idxv], sem).wait()
            pltpu.sync_copy(ncl_ref.at[pl.ds(o, ch)], rowsv)
            pltpu.async_copy(rowsv, c_ref.at[idxv], sem).wait()

    return k


def _stage1n_body(x_ref, flat_ref, wiou_ref, uiou_ref, biou_ref, iu_ref, o_ref):
    out_f = iu_ref.shape[-1]
    iou = (jnp.dot(x_ref[...], wiou_ref[...], preferred_element_type=jnp.float32)
           + jnp.dot(flat_ref[...], uiou_ref[...], preferred_element_type=jnp.float32)
           + biou_ref[...])
    i = jax.nn.sigmoid(iou[:, :out_f])
    o = jax.nn.sigmoid(iou[:, out_f:2 * out_f])
    u = jnp.tanh(iou[:, 2 * out_f:])
    iu_ref[...] = i * u
    o_ref[...] = o


def _stage1e_body(xe_ref, ch_ref, cc_ref, wf_ref, uf_ref, bf_ref, fc_ref):
    f = jax.nn.sigmoid(
        jnp.dot(xe_ref[...], wf_ref[...], preferred_element_type=jnp.float32)
        + jnp.dot(ch_ref[...], uf_ref[...], preferred_element_type=jnp.float32)
        + bf_ref[...])
    fc_ref[...] = f * cc_ref[...]


def _stage2_body(flat_ref, wc_ref, bc_ref, iu_ref, o_ref, h_ref, c_ref):
    cr = jnp.dot(flat_ref[...], wc_ref[...],
                 preferred_element_type=jnp.float32) + bc_ref[...]
    nc = iu_ref[...] + cr
    c_ref[...] = nc
    h_ref[...] = o_ref[...] * jnp.tanh(nc)


def kernel(forest, adjacency, node_order, edge_order, W_iou_w, W_iou_b,
           U_iou_w, W_c_w, W_c_b, W_f_w, W_f_b, U_f_w):
    N, in_f = forest.shape
    out_f = W_f_w.shape[0]
    trip = 3 * out_f
    E = adjacency.shape[0]
    max_it = 4
    bf3 = 3

    parent = adjacency[:, 0]
    child = adjacency[:, 1]
    slot = jnp.clip(adjacency[:, 2] + 1, 0, bf3 - 1)
    valid = (parent >= 0) & (parent < N) & (child >= 0) & (child < N)
    sp = jnp.clip(parent, 0, N - 1)
    sc_ = jnp.clip(child, 0, N - 1)

    wiou_t = W_iou_w.T
    uiou_t = U_iou_w.T
    wc_t = W_c_w.T
    wf_t = W_f_w.T
    uf_t = U_f_w.T
    b_iou = W_iou_b.reshape(1, trip)
    b_c = W_c_b.reshape(1, out_f)
    b_f = W_f_b.reshape(1, out_f)

    node_iter_of_parent = node_order[sp]
    arangeN = jnp.arange(N, dtype=jnp.int32)
    arangeE = jnp.arange(E, dtype=jnp.int32)
    inv_pos = jnp.zeros((N,), jnp.int32)
    active, counts, elists, ecounts = [], [], [], []
    for t in range(max_it):
        mask = node_order == t
        r = jnp.cumsum(mask.astype(jnp.int32)) - 1
        inv_pos = inv_pos + jnp.where(mask, r, 0)
        dest = jnp.where(mask, r, NC)
        active.append(jnp.zeros((NC,), jnp.int32).at[dest].set(
            arangeN, mode="drop"))
        counts.append(jnp.sum(mask.astype(jnp.int32)))
        emask = valid & (edge_order == t) & (node_iter_of_parent == t)
        er = jnp.cumsum(emask.astype(jnp.int32)) - 1
        edest = jnp.where(emask, er, EC)
        elists.append(jnp.zeros((EC,), jnp.int32).at[edest].set(
            arangeE, mode="drop"))
        ecounts.append(jnp.sum(emask.astype(jnp.int32)))

    eids_all = jnp.concatenate(elists)
    cidx_all = sc_[eids_all].reshape(max_it, EC)
    pidx_all = sp[eids_all]
    dest3_all = inv_pos[pidx_all] * bf3 + slot[eids_all]
    ev_all = (jnp.tile(arangeE[:EC], max_it)
              < jnp.repeat(jnp.stack(ecounts), EC))
    dest3_all = jnp.where(ev_all, dest3_all, NC * bf3).reshape(max_it, EC)
    aid_all = jnp.concatenate(active)
    av_all = (jnp.tile(arangeN[:NC], max_it)
              < jnp.repeat(jnp.stack(counts), NC))
    hdest_all = jnp.where(av_all, aid_all, N).reshape(max_it, NC)

    gather_x = _mk_gather1(max_it * NC, N)
    gather_xe = _mk_gather1(max_it * EC, N)
    gather_hc = _mk_gather2(EC)
    update_hc = _mk_update_hc(NC)

    x_all = gather_x(forest, aid_all)
    xe_all = gather_xe(forest, pidx_all)

    g_n = NC // ROWS_N
    g_e = EC // ROWS_E
    full = lambda i: (0, 0)
    blk = lambda i: (i, 0)

    def mk_stage1n(t):
        xoff = t * g_n
        return pl.pallas_call(
            _stage1n_body,
            grid=(g_n,),
            in_specs=[
                pl.BlockSpec((ROWS_N, in_f), lambda i: (xoff + i, 0)),
                pl.BlockSpec((ROWS_N, trip), blk),
                pl.BlockSpec((in_f, trip), full),
                pl.BlockSpec((trip, trip), full),
                pl.BlockSpec((1, trip), full),
            ],
            out_specs=[
                pl.BlockSpec((ROWS_N, out_f), blk),
                pl.BlockSpec((ROWS_N, out_f), blk),
            ],
            out_shape=[
                jax.ShapeDtypeStruct((NC, out_f), jnp.float32),
                jax.ShapeDtypeStruct((NC, out_f), jnp.float32),
            ],
        )

    def mk_stage1e(t):
        xoff = t * g_e
        return pl.pallas_call(
            _stage1e_body,
            grid=(g_e,),
            in_specs=[
                pl.BlockSpec((ROWS_E, in_f), lambda i: (xoff + i, 0)),
                pl.BlockSpec((ROWS_E, out_f), blk),
                pl.BlockSpec((ROWS_E, out_f), blk),
                pl.BlockSpec((in_f, out_f), full),
                pl.BlockSpec((out_f, out_f), full),
                pl.BlockSpec((1, out_f), full),
            ],
            out_specs=pl.BlockSpec((ROWS_E, out_f), blk),
            out_shape=jax.ShapeDtypeStruct((EC, out_f), jnp.float32),
        )

    stage2 = pl.pallas_call(
        _stage2_body,
        grid=(g_n,),
        in_specs=[
            pl.BlockSpec((ROWS_N, trip), blk),
            pl.BlockSpec((trip, out_f), full),
            pl.BlockSpec((1, out_f), full),
            pl.BlockSpec((ROWS_N, out_f), blk),
            pl.BlockSpec((ROWS_N, out_f), blk),
        ],
        out_specs=[
            pl.BlockSpec((ROWS_N, out_f), blk),
            pl.BlockSpec((ROWS_N, out_f), blk),
        ],
        out_shape=[
            jax.ShapeDtypeStruct((NC, out_f), jnp.float32),
            jax.ShapeDtypeStruct((NC, out_f), jnp.float32),
        ],
    )

    h_ref = jax.new_ref(jnp.zeros((N + 16, out_f), jnp.float32))
    c_ref = jax.new_ref(jnp.zeros((N + 16, out_f), jnp.float32))

    for t in range(max_it):
        ch, cc = gather_hc(h_ref, c_ref, cidx_all[t])
        dest3 = dest3_all[t]
        flat_h = (jnp.zeros((NC * bf3, out_f), jnp.float32)
                  .at[dest3].add(ch, mode="drop").reshape(NC, trip))
        iu, o = mk_stage1n(t)(x_all, flat_h, wiou_t, uiou_t, b_iou)
        fc = mk_stage1e(t)(xe_all, ch, cc, wf_t, uf_t, b_f)
        flat_fc = (jnp.zeros((NC * bf3, out_f), jnp.float32)
                   .at[dest3].add(fc, mode="drop").reshape(NC, trip))
        nh, ncell = stage2(flat_fc, wc_t, b_c, iu, o)
        update_hc(nh, ncell, hdest_all[t], h_ref, c_ref)

    return h_ref[...][:N]
